# Initial kernel scaffold; baseline (speedup 1.0000x reference)
import jax
import jax.numpy as jnp
from jax.experimental import pallas as pl


def kernel(position_ids, table):
    raise NotImplementedError("write your pallas kernel here")


# speedup vs baseline: 2.1876x; 2.1876x over previous
import functools

import jax
import jax.numpy as jnp
from jax import lax
from jax.experimental import pallas as pl
from jax.experimental.pallas import tpu as pltpu
from jax.experimental.pallas import tpu_sc as plsc

D = 1024
NC = 2
NS = 16
NW = NC * NS
CHUNK = 64


def kernel(position_ids, table):
    batch, seq = position_ids.shape
    total = batch * seq
    per_w = total // NW
    n_chunk = per_w // CHUNK
    idx = position_ids.reshape(total)

    mesh = plsc.VectorSubcoreMesh(core_axis_name="c", subcore_axis_name="s")

    @functools.partial(
        pl.kernel,
        out_type=jax.ShapeDtypeStruct((total, D), jnp.float32),
        mesh=mesh,
        scratch_types=[
            pltpu.VMEM((per_w,), jnp.int32),
            pltpu.VMEM((CHUNK, D), jnp.float32),
            pltpu.SemaphoreType.DMA,
        ],
    )
    def gather_kernel(idx_hbm, table_hbm, out_hbm, idx_v, rows_v, sem):
        wid = lax.axis_index("s") * NC + lax.axis_index("c")
        base = wid * per_w
        pltpu.sync_copy(idx_hbm.at[pl.ds(base, per_w)], idx_v)

        @pl.loop(0, n_chunk)
        def _(i):
            off = i * CHUNK
            pltpu.async_copy(
                table_hbm.at[idx_v.at[pl.ds(off, CHUNK)]], rows_v, sem
            ).wait()
            pltpu.sync_copy(rows_v, out_hbm.at[pl.ds(base + off, CHUNK)])

    out = gather_kernel(idx, table)
    return out.reshape(batch, seq, D)


# speedup vs baseline: 2.3869x; 1.0911x over previous
import functools

import jax
import jax.numpy as jnp
from jax import lax
from jax.experimental import pallas as pl
from jax.experimental.pallas import tpu as pltpu
from jax.experimental.pallas import tpu_sc as plsc

D = 1024
NC = 2
NS = 16
NW = NC * NS
CHUNK = 32


def kernel(position_ids, table):
    batch, seq = position_ids.shape
    total = batch * seq
    per_w = total // NW
    n_chunk = per_w // CHUNK
    n_pair = n_chunk // 2
    idx = position_ids.reshape(total)

    mesh = plsc.VectorSubcoreMesh(core_axis_name="c", subcore_axis_name="s")

    @functools.partial(
        pl.kernel,
        out_type=jax.ShapeDtypeStruct((total, D), jnp.float32),
        mesh=mesh,
        scratch_types=[
            pltpu.VMEM((per_w,), jnp.int32),
            pltpu.VMEM((CHUNK, D), jnp.float32),
            pltpu.VMEM((CHUNK, D), jnp.float32),
            pltpu.SemaphoreType.DMA,
            pltpu.SemaphoreType.DMA,
            pltpu.SemaphoreType.DMA,
            pltpu.SemaphoreType.DMA,
        ],
    )
    def gather_kernel(idx_hbm, table_hbm, out_hbm, idx_v, buf0, buf1,
                      g0, g1, w0, w1):
        wid = lax.axis_index("s") * NC + lax.axis_index("c")
        base = wid * per_w
        pltpu.sync_copy(idx_hbm.at[pl.ds(base, per_w)], idx_v)

        def gather(c, buf, sem):
            return pltpu.make_async_copy(
                table_hbm.at[idx_v.at[pl.ds(c * CHUNK, CHUNK)]], buf, sem)

        def write(c, buf, sem):
            return pltpu.make_async_copy(
                buf, out_hbm.at[pl.ds(base + c * CHUNK, CHUNK)], sem)

        gather(0, buf0, g0).start()

        @pl.loop(0, n_pair)
        def _(j):
            i = 2 * j

            @pl.when(j > 0)
            def _():
                write(i - 1, buf1, w1).wait()

            gather(i + 1, buf1, g1).start()
            gather(i, buf0, g0).wait()
            write(i, buf0, w0).start()

            @pl.when(j < n_pair - 1)
            def _():
                write(i, buf0, w0).wait()
                gather(i + 2, buf0, g0).start()

            gather(i + 1, buf1, g1).wait()
            write(i + 1, buf1, w1).start()

        write(n_chunk - 2, buf0, w0).wait()
        write(n_chunk - 1, buf1, w1).wait()

    out = gather_kernel(idx, table)
    return out.reshape(batch, seq, D)


# speedup vs baseline: 2.3884x; 1.0006x over previous
import functools

import jax
import jax.numpy as jnp
from jax import lax
from jax.experimental import pallas as pl
from jax.experimental.pallas import tpu as pltpu
from jax.experimental.pallas import tpu_sc as plsc

D = 1024
NC = 2
NS = 16
NW = NC * NS
CHUNK = 32
NBUF = 3


def kernel(position_ids, table):
    batch, seq = position_ids.shape
    total = batch * seq
    per_w = total // NW
    n_chunk = per_w // CHUNK
    main_iters = n_chunk // NBUF
    idx = position_ids.reshape(total)

    mesh = plsc.VectorSubcoreMesh(core_axis_name="c", subcore_axis_name="s")

    @functools.partial(
        pl.kernel,
        out_type=jax.ShapeDtypeStruct((total, D), jnp.float32),
        mesh=mesh,
        scratch_types=(
            [pltpu.VMEM((per_w,), jnp.int32)]
            + [pltpu.VMEM((CHUNK, D), jnp.float32) for _ in range(NBUF)]
            + [pltpu.SemaphoreType.DMA for _ in range(2 * NBUF)]
        ),
    )
    def gather_kernel(idx_hbm, table_hbm, out_hbm, idx_v, *rest):
        bufs = rest[:NBUF]
        gsem = rest[NBUF:2 * NBUF]
        wsem = rest[2 * NBUF:]
        wid = lax.axis_index("s") * NC + lax.axis_index("c")
        base = wid * per_w
        pltpu.sync_copy(idx_hbm.at[pl.ds(base, per_w)], idx_v)

        def gather(c, b):
            return pltpu.make_async_copy(
                table_hbm.at[idx_v.at[pl.ds(c * CHUNK, CHUNK)]],
                bufs[b], gsem[b])

        def write(c, b):
            return pltpu.make_async_copy(
                bufs[b], out_hbm.at[pl.ds(base + c * CHUNK, CHUNK)], wsem[b])

        for b in range(NBUF):
            gather(b, b).start()

        @pl.loop(0, main_iters)
        def _(j):
            i0 = j * NBUF
            for b in range(NBUF):
                i = i0 + b
                gather(i, b).wait()
                write(i, b).start()

                @pl.when(i + NBUF < n_chunk)
                def _():
                    write(i, b).wait()
                    gather(i + NBUF, b).start()

        for i in range(main_iters * NBUF, n_chunk):
            b = i % NBUF
            gather(i, b).wait()
            write(i, b).start()
        for i in range(n_chunk - NBUF, n_chunk):
            write(i, i % NBUF).wait()

    out = gather_kernel(idx, table)
    return out.reshape(batch, seq, D)
